# Initial kernel scaffold; baseline (speedup 1.0000x reference)
import jax
import jax.numpy as jnp
from jax.experimental import pallas as pl


def kernel(node_features, node_attributes, edge_sh, edge_src, edge_dst, edge_embedded, W_sc, W_lin1, W_fc0, W_fc1, W_fc2, W_lin2):
    raise NotImplementedError("write your pallas kernel here")


# speedup vs baseline: 2.5818x; 2.5818x over previous
import functools
import math

import jax
import jax.numpy as jnp
from jax import lax
from jax.experimental import pallas as pl
from jax.experimental.pallas import tpu as pltpu
from jax.experimental.pallas import tpu_sc as plsc

_NC = 2
_NS = 16
_NW = _NC * _NS
_L = 16
_EB = 128


def _node_stage(nf, na, w_sc2, w_lin1, block_n):
  n, d = nf.shape
  inv = 1.0 / math.sqrt(d)

  def body(nf_ref, na_ref, wsc_ref, wl1_ref, h_ref, sc_ref):
    nf_b = nf_ref[...]
    h_ref[...] = jnp.dot(nf_b, wl1_ref[...],
                         preferred_element_type=jnp.float32) * inv
    sc_ref[...] = na_ref[...] * (
        jnp.dot(nf_b, wsc_ref[...], preferred_element_type=jnp.float32) * inv)

  grid = (n // block_n,)
  return pl.pallas_call(
      body,
      grid=grid,
      in_specs=[
          pl.BlockSpec((block_n, d), lambda i: (i, 0)),
          pl.BlockSpec((block_n, 1), lambda i: (i, 0)),
          pl.BlockSpec((d, d), lambda i: (0, 0)),
          pl.BlockSpec((d, d), lambda i: (0, 0)),
      ],
      out_specs=[
          pl.BlockSpec((block_n, d), lambda i: (i, 0)),
          pl.BlockSpec((block_n, d), lambda i: (i, 0)),
      ],
      out_shape=[
          jax.ShapeDtypeStruct((n, d), jnp.float32),
          jax.ShapeDtypeStruct((n, d), jnp.float32),
      ],
  )(nf, na, w_sc2, w_lin1)


def _edge_weight_stage(ee, sh, w0, w1, w2, block_e):
  e, nb = ee.shape
  h_dim = w0.shape[1]
  d = w2.shape[1]
  inv_nb = 1.0 / math.sqrt(nb)
  inv_h = 1.0 / math.sqrt(h_dim)

  def body(ee_ref, sh_ref, w0_ref, w1_ref, w2_ref, out_ref):
    x = jax.nn.swish(jnp.dot(ee_ref[...], w0_ref[...],
                             preferred_element_type=jnp.float32) * inv_nb)
    x = jax.nn.swish(jnp.dot(x, w1_ref[...],
                             preferred_element_type=jnp.float32) * inv_h)
    out_ref[...] = (jnp.dot(x, w2_ref[...],
                            preferred_element_type=jnp.float32) * inv_h
                    ) * sh_ref[...]

  grid = (e // block_e,)
  return pl.pallas_call(
      body,
      grid=grid,
      in_specs=[
          pl.BlockSpec((block_e, nb), lambda i: (i, 0)),
          pl.BlockSpec((block_e, 1), lambda i: (i, 0)),
          pl.BlockSpec((nb, h_dim), lambda i: (0, 0)),
          pl.BlockSpec((h_dim, h_dim), lambda i: (0, 0)),
          pl.BlockSpec((h_dim, d), lambda i: (0, 0)),
      ],
      out_specs=pl.BlockSpec((block_e, d), lambda i: (i, 0)),
      out_shape=jax.ShapeDtypeStruct((e, d), jnp.float32),
  )(ee, sh, w0, w1, w2)


def _make_sc_stage(n_pad, d, nblk):
  rows_per_tile = n_pad // _NS
  chunks = []
  off = 0
  while off < rows_per_tile:
    cnt = min(_EB, rows_per_tile - off)
    chunks.append((off, cnt))
    off += cnt
  nfull = nblk // _NW
  rem = nblk % _NW
  mesh = plsc.VectorSubcoreMesh(core_axis_name="c", subcore_axis_name="s")

  @functools.partial(
      pl.kernel,
      out_type=jax.ShapeDtypeStruct((_NC, n_pad, d), jnp.float32),
      mesh=mesh,
      scratch_types=[
          pltpu.VMEM((_EB,), jnp.int32),
          pltpu.VMEM((_EB,), jnp.int32),
          pltpu.VMEM((_EB, d), jnp.float32),
          pltpu.VMEM((_EB, d), jnp.float32),
          pltpu.VMEM_SHARED((n_pad, d), jnp.float32),
      ],
  )
  def sc_k(h_hbm, w_hbm, src_hbm, dst_hbm, out_hbm,
           src_v, dst_v, rows_v, w_v, agg_sh):
    c = lax.axis_index("c")
    s = lax.axis_index("s")
    wid = s * _NC + c
    base = s * rows_per_tile

    zero = jnp.zeros((_L,), jnp.float32)

    def zrow(i, carry):
      for j in range(d // _L):
        rows_v[i, pl.ds(j * _L, _L)] = zero
      return carry

    lax.fori_loop(0, _EB, zrow, 0)
    for coff, cnt in chunks:
      pltpu.sync_copy(rows_v.at[pl.ds(0, cnt)],
                      agg_sh.at[pl.ds(base + coff, cnt)])
    plsc.subcore_barrier()

    trips = nfull + jnp.where(wid < rem, 1, 0)

    def eblock(j, carry):
      b = wid + j * _NW
      pltpu.sync_copy(src_hbm.at[pl.ds(b * _EB, _EB)], src_v)
      pltpu.sync_copy(dst_hbm.at[pl.ds(b * _EB, _EB)], dst_v)
      pltpu.sync_copy(w_hbm.at[b], w_v)
      pltpu.sync_copy(h_hbm.at[src_v], rows_v)

      def mrow(i, cc):
        for j2 in range(d // _L):
          sl = pl.ds(j2 * _L, _L)
          rows_v[i, sl] = rows_v[i, sl] * w_v[i, sl]
        return cc

      lax.fori_loop(0, _EB, mrow, 0)
      pltpu.sync_copy(rows_v, agg_sh.at[dst_v], add=True)
      return carry

    lax.fori_loop(0, trips, eblock, 0)
    plsc.subcore_barrier()

    for coff, cnt in chunks:
      pltpu.sync_copy(agg_sh.at[pl.ds(base + coff, cnt)],
                      rows_v.at[pl.ds(0, cnt)])
      pltpu.sync_copy(rows_v.at[pl.ds(0, cnt)],
                      out_hbm.at[c, pl.ds(base + coff, cnt)])

  return sc_k


def _final_stage(aggs, sc, w_lin2, n_neighbors, block_n, n_out):
  _, _, d = aggs.shape
  n = n_out
  scale = 1.0 / (n_neighbors * math.sqrt(d))

  def body(agg_ref, sc_ref, wl2_ref, out_ref):
    a = agg_ref[0] + agg_ref[1]
    h2 = jnp.dot(a, wl2_ref[...],
                 preferred_element_type=jnp.float32) * scale + sc_ref[...]
    out_ref[...] = jax.nn.swish(h2)

  grid = (n // block_n,)
  return pl.pallas_call(
      body,
      grid=grid,
      in_specs=[
          pl.BlockSpec((_NC, block_n, d), lambda i: (0, i, 0)),
          pl.BlockSpec((block_n, d), lambda i: (i, 0)),
          pl.BlockSpec((d, d), lambda i: (0, 0)),
      ],
      out_specs=pl.BlockSpec((block_n, d), lambda i: (i, 0)),
      out_shape=jax.ShapeDtypeStruct((n, d), jnp.float32),
  )(aggs, sc, w_lin2)


def kernel(node_features, node_attributes, edge_sh, edge_src, edge_dst,
           edge_embedded, W_sc, W_lin1, W_fc0, W_fc1, W_fc2, W_lin2):
  n, d = node_features.shape
  e = edge_src.shape[0]

  h, sc = _node_stage(node_features, node_attributes, W_sc[:, 0, :], W_lin1,
                      block_n=1000)

  w_edge = _edge_weight_stage(edge_embedded, edge_sh, W_fc0, W_fc1, W_fc2,
                              block_e=8000)

  e_pad = ((e + _EB - 1) // _EB) * _EB
  src = edge_src.astype(jnp.int32)
  dst = edge_dst.astype(jnp.int32)
  if e_pad != e:
    pad = e_pad - e
    src = jnp.concatenate([src, jnp.zeros((pad,), jnp.int32)])
    dst = jnp.concatenate([dst, jnp.zeros((pad,), jnp.int32)])
    w_edge = jnp.concatenate([w_edge, jnp.zeros((pad, d), jnp.float32)])
  nblk = e_pad // _EB

  n_pad = ((n + _NS * 8 - 1) // (_NS * 8)) * (_NS * 8)

  aggs = _make_sc_stage(n_pad, d, nblk)(
      h,
      w_edge.reshape(nblk, _EB, d),
      src,
      dst,
  )

  return _final_stage(aggs, sc, W_lin2, 32.0, block_n=1000, n_out=n)
